# Initial kernel scaffold; baseline (speedup 1.0000x reference)
import jax
import jax.numpy as jnp
from jax.experimental import pallas as pl


def kernel(node_s, node_v, edge_s, edge_v, dist, vctr_norm, edge_index, W_nn, b_nn, W_ep, b_ep, gm_W1, gm_b1, gm_Wo, gm_bo, gm_Wg, gm_bg, W_ev, b_ev, W_nvout, W_nvch, b_nvch, W_nvproj, W_nsproj, b_nsproj, ln_g, ln_b, cn_scale):
    raise NotImplementedError("write your pallas kernel here")


# speedup vs baseline: 10.0798x; 10.0798x over previous
import functools

import jax
import jax.numpy as jnp
from jax import lax
from jax.experimental import pallas as pl
from jax.experimental.pallas import tpu as pltpu
from jax.experimental.pallas import tpu_sc as plsc

N = 10000
E = 160000
F = 128
CUTOFF = 5.0

_EDGE_BLK = 2000
_NODE_BLK = 1000
_CHUNK = 128
_NCHUNKS = E // _CHUNK
_NW = 32
_GITER = (_NCHUNKS + _NW - 1) // _NW
_SITER = (_NCHUNKS + 15) // 16
_TROWS = 624
_TREM = N - 16 * _TROWS
_ZROWS = 104
_GC = 64
_NG = (4 * F) // _GC


def _silu(x):
    return x * jax.nn.sigmoid(x)


def _sc_gather(src, dst, ns_tab, nv_tab):
    mesh = plsc.VectorSubcoreMesh(core_axis_name="c", subcore_axis_name="s")

    @functools.partial(
        pl.kernel,
        mesh=mesh,
        out_type=(
            jax.ShapeDtypeStruct((E, F), jnp.float32),
            jax.ShapeDtypeStruct((E, F), jnp.float32),
            jax.ShapeDtypeStruct((E, 3 * F), jnp.float32),
        ),
        scratch_types=(
            pltpu.VMEM((_CHUNK,), jnp.int32),
            pltpu.VMEM((_CHUNK,), jnp.int32),
            pltpu.VMEM((_CHUNK, F), jnp.float32),
            pltpu.VMEM((_CHUNK, F), jnp.float32),
            pltpu.VMEM((_CHUNK, 3 * F), jnp.float32),
            pltpu.SemaphoreType.DMA,
        ),
    )
    def k(src_h, dst_h, ns_h, nv_h, gs_h, gd_h, gv_h,
          srcv, dstv, bs, bd, bv, sem):
        c = lax.axis_index("c")
        s = lax.axis_index("s")
        w = s * 2 + c

        def body(i, carry):
            j = i * _NW + w

            @pl.when(j < _NCHUNKS)
            def _():
                e0 = j * _CHUNK
                pltpu.sync_copy(src_h.at[pl.ds(e0, _CHUNK)], srcv)
                pltpu.sync_copy(dst_h.at[pl.ds(e0, _CHUNK)], dstv)
                c1 = pltpu.async_copy(ns_h.at[srcv], bs, sem)
                c2 = pltpu.async_copy(ns_h.at[dstv], bd, sem)
                c3 = pltpu.async_copy(nv_h.at[srcv], bv, sem)
                c1.wait()
                c2.wait()
                c3.wait()
                pltpu.sync_copy(bs, gs_h.at[pl.ds(e0, _CHUNK)])
                pltpu.sync_copy(bd, gd_h.at[pl.ds(e0, _CHUNK)])
                pltpu.sync_copy(bv, gv_h.at[pl.ds(e0, _CHUNK)])

            return carry

        lax.fori_loop(0, _GITER, body, 0)

    return k(src, dst, ns_tab, nv_tab)


def _edge_body(gs_r, gd_r, gv_r, es_r, ev_r, d_r, vn_r,
               wnn_r, bnn_r, wep_r, bep_r, w1_r, b1_r, wo_r, bo_r, wg_r, bg_r,
               wev_r, bev_r, eso_r, evo_r, u0_r, u1_r, u2_r, u3_r):
    d = d_r[...]
    cut = 0.5 * (jnp.cos(jnp.pi * d / CUTOFF) + 1.0) * (d < CUTOFF).astype(jnp.float32)
    wnn = wnn_r[...]
    nn_feat = gs_r[...] @ wnn[:F] + gd_r[...] @ wnn[F:] + bnn_r[...]
    em = nn_feat * (es_r[...] @ wep_r[...] + bep_r[...])
    h = _silu(em @ w1_r[...] + b1_r[...])
    es_u = (h @ wo_r[...] + bo_r[...]) * jax.nn.sigmoid(h @ wg_r[...] + bg_r[...]) * cut
    eso_r[...] = es_u + es_r[...]
    vc = es_u @ wev_r[...] + bev_r[...]
    gvv = gv_r[...]
    evv = ev_r[...]
    vn = vn_r[...]
    ch_n = vc[:, :F]
    ch_e = vc[:, F:2 * F]
    ch_r = vc[:, 2 * F:]
    comps = []
    for kk in range(3):
        comp = (gvv[:, kk * F:(kk + 1) * F] * ch_n
                + evv[:, kk * F:(kk + 1) * F] * ch_e
                + vn[:, kk:kk + 1] * ch_r)
        comps.append(comp * cut)
    ev_u = jnp.concatenate(comps, axis=1)
    evo_r[...] = ev_u + evv
    u0_r[...] = es_u
    u1_r[...] = comps[0]
    u2_r[...] = comps[1]
    u3_r[...] = comps[2]


def _edge_tc(gs, gd, gv, es, ev, dist, vctr,
             W_nn, b_nn, W_ep, b_ep, gm_W1, gm_b1, gm_Wo, gm_bo, gm_Wg, gm_bg,
             W_ev, b_ev):
    nblk = E // _EDGE_BLK
    eb = lambda i: (i, 0)
    wb = lambda i: (0, 0)
    return pl.pallas_call(
        _edge_body,
        grid=(nblk,),
        in_specs=[
            pl.BlockSpec((_EDGE_BLK, F), eb),
            pl.BlockSpec((_EDGE_BLK, F), eb),
            pl.BlockSpec((_EDGE_BLK, 3 * F), eb),
            pl.BlockSpec((_EDGE_BLK, F), eb),
            pl.BlockSpec((_EDGE_BLK, 3 * F), eb),
            pl.BlockSpec((_EDGE_BLK, 1), eb),
            pl.BlockSpec((_EDGE_BLK, 3), eb),
            pl.BlockSpec((2 * F, F), wb),
            pl.BlockSpec((1, F), wb),
            pl.BlockSpec((F, F), wb),
            pl.BlockSpec((1, F), wb),
            pl.BlockSpec((F, F), wb),
            pl.BlockSpec((1, F), wb),
            pl.BlockSpec((F, F), wb),
            pl.BlockSpec((1, F), wb),
            pl.BlockSpec((F, F), wb),
            pl.BlockSpec((1, F), wb),
            pl.BlockSpec((F, 3 * F), wb),
            pl.BlockSpec((1, 3 * F), wb),
        ],
        out_specs=[
            pl.BlockSpec((_EDGE_BLK, F), eb),
            pl.BlockSpec((_EDGE_BLK, 3 * F), eb),
            pl.BlockSpec((_EDGE_BLK, F), eb),
            pl.BlockSpec((_EDGE_BLK, F), eb),
            pl.BlockSpec((_EDGE_BLK, F), eb),
            pl.BlockSpec((_EDGE_BLK, F), eb),
        ],
        out_shape=[
            jax.ShapeDtypeStruct((E, F), jnp.float32),
            jax.ShapeDtypeStruct((E, 3 * F), jnp.float32),
            jax.ShapeDtypeStruct((E, F), jnp.float32),
            jax.ShapeDtypeStruct((E, F), jnp.float32),
            jax.ShapeDtypeStruct((E, F), jnp.float32),
            jax.ShapeDtypeStruct((E, F), jnp.float32),
        ],
    )(gs, gd, gv, es, ev, dist, vctr,
      W_nn, b_nn.reshape(1, F), W_ep, b_ep.reshape(1, F),
      gm_W1, gm_b1.reshape(1, F), gm_Wo, gm_bo.reshape(1, F),
      gm_Wg, gm_bg.reshape(1, F), W_ev, b_ev.reshape(1, 3 * F))


def _sc_scatter(u0, u1, u2, u3, dst, zrows, ones):
    mesh = plsc.VectorSubcoreMesh(core_axis_name="c", subcore_axis_name="s")

    @functools.partial(
        pl.kernel,
        mesh=mesh,
        out_type=tuple(jax.ShapeDtypeStruct((N, F), jnp.float32)
                       for _ in range(4)) + (
            jax.ShapeDtypeStruct((2 * N, F), jnp.float32),),
        scratch_types=(
            pltpu.VMEM((_CHUNK,), jnp.int32),
            pltpu.VMEM((_CHUNK, F), jnp.float32),
            pltpu.VMEM((_CHUNK, F), jnp.float32),
            pltpu.VMEM_SHARED((N, F), jnp.float32),
            pltpu.SemaphoreType.DMA,
        ),
    )
    def k(u0_h, u1_h, u2_h, u3_h, dst_h, z_h, o_h,
          a0_h, a1_h, a2_h, a3_h, deg_h,
          dstv, valbuf, onesbuf, acc, sem):
        c = lax.axis_index("c")
        s = lax.axis_index("s")
        w = s * 2 + c
        pltpu.sync_copy(o_h, onesbuf)

        u_hs = (u0_h, u1_h, u2_h, u3_h)
        a_hs = (a0_h, a1_h, a2_h, a3_h)

        for p in range(3):
            pltpu.sync_copy(z_h, acc.at[pl.ds(s * _TROWS, _TROWS)])

            @pl.when(s == 0)
            def _():
                pltpu.sync_copy(z_h.at[pl.ds(0, _TREM)],
                                acc.at[pl.ds(16 * _TROWS, _TREM)])

            plsc.subcore_barrier()

            if p < 2:
                def body(i, carry):
                    j = i * 16 + s

                    @pl.when(j < _NCHUNKS)
                    def _():
                        e0 = j * _CHUNK
                        pltpu.sync_copy(dst_h.at[pl.ds(e0, _CHUNK)], dstv)

                        @pl.when(c == 0)
                        def _():
                            pltpu.sync_copy(u_hs[p].at[pl.ds(e0, _CHUNK)],
                                            valbuf)

                        @pl.when(c == 1)
                        def _():
                            pltpu.sync_copy(u_hs[2 + p].at[pl.ds(e0, _CHUNK)],
                                            valbuf)

                        pltpu.sync_copy(valbuf, acc.at[dstv], add=True)

                    return carry

                lax.fori_loop(0, _SITER, body, 0)
                plsc.subcore_barrier()

                @pl.when(c == 0)
                def _():
                    pltpu.sync_copy(acc.at[pl.ds(s * _TROWS, _TROWS)],
                                    a_hs[p].at[pl.ds(s * _TROWS, _TROWS)])

                    @pl.when(s == 0)
                    def _():
                        pltpu.sync_copy(acc.at[pl.ds(16 * _TROWS, _TREM)],
                                        a_hs[p].at[pl.ds(16 * _TROWS, _TREM)])

                @pl.when(c == 1)
                def _():
                    pltpu.sync_copy(acc.at[pl.ds(s * _TROWS, _TROWS)],
                                    a_hs[2 + p].at[pl.ds(s * _TROWS, _TROWS)])

                    @pl.when(s == 0)
                    def _():
                        pltpu.sync_copy(acc.at[pl.ds(16 * _TROWS, _TREM)],
                                        a_hs[2 + p].at[pl.ds(16 * _TROWS, _TREM)])

                plsc.subcore_barrier()
            else:
                def dbody(i, carry):
                    j = i * _NW + w

                    @pl.when(j < _NCHUNKS)
                    def _():
                        e0 = j * _CHUNK
                        pltpu.sync_copy(dst_h.at[pl.ds(e0, _CHUNK)], dstv)
                        pltpu.sync_copy(onesbuf, acc.at[dstv], add=True)

                    return carry

                lax.fori_loop(0, _GITER, dbody, 0)
                plsc.subcore_barrier()
                pltpu.sync_copy(
                    acc.at[pl.ds(s * _TROWS, _TROWS)],
                    deg_h.at[pl.ds(c * N + s * _TROWS, _TROWS)])

                @pl.when(s == 0)
                def _():
                    pltpu.sync_copy(
                        acc.at[pl.ds(16 * _TROWS, _TREM)],
                        deg_h.at[pl.ds(c * N + 16 * _TROWS, _TREM)])

    return k(u0, u1, u2, u3, dst, zrows, ones)


def _node_body(a0_r, a1_r, a2_r, a3_r, dega_r, degb_r, ns_r, nv_r,
               wvo_r, wvc_r, bvc_r, wvp_r, wsp_r, bsp_r, lg_r, lb_r, cns_r,
               nso_r, nvo_r):
    deg = dega_r[:, 0:1] + degb_r[:, 0:1]
    denom = jnp.maximum(deg, 1.0)
    n_es = a0_r[...] / denom
    evs = [a1_r[...] / denom, a2_r[...] / denom, a3_r[...] / denom]
    wvo = wvo_r[...]
    outs = [ev @ wvo for ev in evs]
    o3s = [o[:, 2 * F:] for o in outs]
    o3n = jnp.sqrt(o3s[0] ** 2 + o3s[1] ** 2 + o3s[2] ** 2)
    wvc = wvc_r[...]
    v_ch = n_es @ wvc[:F] + o3n @ wvc[F:] + bvc_r[...]
    us = [outs[kk][:, :F] * v_ch + outs[kk][:, F:2 * F] for kk in range(3)]
    wvp = wvp_r[...]
    ps = [u @ wvp for u in us]
    sp = _silu(n_es @ wsp_r[...] + bsp_r[...])
    nv_dot = (ps[0][:, :F] * ps[0][:, F:]
              + ps[1][:, :F] * ps[1][:, F:]
              + ps[2][:, :F] * ps[2][:, F:])
    n_s_u = nv_dot * sp[:, :F] + sp[:, F:]
    ns_res = n_s_u + ns_r[...]
    mu = jnp.mean(ns_res, axis=-1, keepdims=True)
    var = jnp.mean((ns_res - mu) ** 2, axis=-1, keepdims=True)
    nso_r[...] = (ns_res - mu) / jnp.sqrt(var + 1e-5) * lg_r[...] + lb_r[...]
    nvv = nv_r[...]
    res = [us[kk] + nvv[:, kk * F:(kk + 1) * F] for kk in range(3)]
    vn = jnp.sqrt(res[0] ** 2 + res[1] ** 2 + res[2] ** 2)
    cns = cns_r[...]
    nvo_r[...] = jnp.concatenate([r / (vn + 1e-8) * cns for r in res], axis=1)


def _node_tc(a0, a1, a2, a3, dega, degb, ns, nv, W_nvout, W_nvch, b_nvch,
             W_nvproj, W_nsproj, b_nsproj, ln_g, ln_b, cn_scale):
    nblk = N // _NODE_BLK
    nb = lambda i: (i, 0)
    wb = lambda i: (0, 0)
    return pl.pallas_call(
        _node_body,
        grid=(nblk,),
        in_specs=[
            pl.BlockSpec((_NODE_BLK, F), nb),
            pl.BlockSpec((_NODE_BLK, F), nb),
            pl.BlockSpec((_NODE_BLK, F), nb),
            pl.BlockSpec((_NODE_BLK, F), nb),
            pl.BlockSpec((_NODE_BLK, F), nb),
            pl.BlockSpec((_NODE_BLK, F), nb),
            pl.BlockSpec((_NODE_BLK, F), nb),
            pl.BlockSpec((_NODE_BLK, 3 * F), nb),
            pl.BlockSpec((F, 3 * F), wb),
            pl.BlockSpec((2 * F, F), wb),
            pl.BlockSpec((1, F), wb),
            pl.BlockSpec((F, 2 * F), wb),
            pl.BlockSpec((F, 2 * F), wb),
            pl.BlockSpec((1, 2 * F), wb),
            pl.BlockSpec((1, F), wb),
            pl.BlockSpec((1, F), wb),
            pl.BlockSpec((1, F), wb),
        ],
        out_specs=[
            pl.BlockSpec((_NODE_BLK, F), nb),
            pl.BlockSpec((_NODE_BLK, 3 * F), nb),
        ],
        out_shape=[
            jax.ShapeDtypeStruct((N, F), jnp.float32),
            jax.ShapeDtypeStruct((N, 3 * F), jnp.float32),
        ],
    )(a0, a1, a2, a3, dega, degb, ns, nv, W_nvout, W_nvch,
      b_nvch.reshape(1, F), W_nvproj,
      W_nsproj, b_nsproj.reshape(1, 2 * F), ln_g.reshape(1, F),
      ln_b.reshape(1, F), cn_scale.reshape(1, F))


def kernel(node_s, node_v, edge_s, edge_v, dist, vctr_norm, edge_index,
           W_nn, b_nn, W_ep, b_ep, gm_W1, gm_b1, gm_Wo, gm_bo, gm_Wg, gm_bg,
           W_ev, b_ev, W_nvout, W_nvch, b_nvch, W_nvproj, W_nsproj, b_nsproj,
           ln_g, ln_b, cn_scale):
    src = edge_index[0]
    dst = edge_index[1]
    ns = node_s.reshape(N, F)
    nv = node_v.reshape(N, 3 * F)
    es = edge_s.reshape(E, F)
    ev = edge_v.reshape(E, 3 * F)

    gs, gd, gv = _sc_gather(src, dst, ns, nv)
    eso, evo, u0, u1, u2, u3 = _edge_tc(gs, gd, gv, es, ev, dist, vctr_norm,
                                        W_nn, b_nn, W_ep, b_ep, gm_W1, gm_b1,
                                        gm_Wo, gm_bo, gm_Wg, gm_bg, W_ev, b_ev)
    zrows = jnp.zeros((_TROWS, F), jnp.float32)
    ones = jnp.ones((_CHUNK, F), jnp.float32)
    a0, a1, a2, a3, deg2 = _sc_scatter(u0, u1, u2, u3, dst, zrows, ones)
    nso, nvo = _node_tc(a0, a1, a2, a3, deg2[:N], deg2[N:], ns, nv, W_nvout,
                        W_nvch, b_nvch, W_nvproj, W_nsproj, b_nsproj,
                        ln_g, ln_b, cn_scale)

    return (nso.reshape(N, 1, F), nvo.reshape(N, 3, F),
            eso.reshape(E, 1, F), evo.reshape(E, 3, F))
